# Initial kernel scaffold; baseline (speedup 1.0000x reference)
import jax
import jax.numpy as jnp
from jax.experimental import pallas as pl


def kernel(x, edge_index, W1_l, W1_r, b1, W2_l, W2_r, b2):
    raise NotImplementedError("write your pallas kernel here")


# speedup vs baseline: 8.1066x; 8.1066x over previous
import functools

import jax
import jax.numpy as jnp
from jax import lax
from jax.experimental import pallas as pl
from jax.experimental.pallas import tpu as pltpu
from jax.experimental.pallas import tpu_sc as plsc

NC = 2
NS = 16
NW = NC * NS
CHUNK = 128
CNTW = 16


def _seg_sum_sc(n_pad, rows_per_tile, feat, chunks, with_count):
    out_type = [jax.ShapeDtypeStruct((NC, n_pad, feat), jnp.float32)]
    scratch = [
        pltpu.VMEM((chunks, CHUNK), jnp.int32),
        pltpu.VMEM((chunks, CHUNK), jnp.int32),
        pltpu.VMEM((CHUNK, feat), jnp.float32),
        pltpu.VMEM_SHARED((n_pad, feat), jnp.float32),
        pltpu.SemaphoreType.DMA,
    ]
    if with_count:
        out_type.append(jax.ShapeDtypeStruct((NC, n_pad, CNTW), jnp.float32))
        scratch += [
            pltpu.VMEM((CHUNK, CNTW), jnp.float32),
            pltpu.VMEM_SHARED((n_pad, CNTW), jnp.float32),
        ]

    mesh = plsc.VectorSubcoreMesh(core_axis_name="c", subcore_axis_name="s")

    def body(*refs):
        if with_count:
            (src_h, dst_h, tab_h, zf_h, zc_h, ones_h, out_h, cnt_h,
             src_v, dst_v, rows_v, acc_sh, sem, ones_v, cnt_sh) = refs
        else:
            (src_h, dst_h, tab_h, zf_h, out_h,
             src_v, dst_v, rows_v, acc_sh, sem) = refs
        cid = lax.axis_index("c")
        sid = lax.axis_index("s")
        wid = sid * NC + cid
        base = sid * rows_per_tile
        pltpu.sync_copy(zf_h, acc_sh.at[pl.ds(base, rows_per_tile)])
        if with_count:
            pltpu.sync_copy(zc_h, cnt_sh.at[pl.ds(base, rows_per_tile)])
            pltpu.sync_copy(ones_h, ones_v)
        pltpu.sync_copy(src_h.at[wid], src_v)
        pltpu.sync_copy(dst_h.at[wid], dst_v)
        plsc.subcore_barrier()

        def step(j, carry):
            pltpu.async_copy(tab_h.at[src_v.at[j]], rows_v, sem).wait()
            pltpu.sync_copy(rows_v, acc_sh.at[dst_v.at[j]], add=True)
            if with_count:
                pltpu.sync_copy(ones_v, cnt_sh.at[dst_v.at[j]], add=True)
            return carry

        lax.fori_loop(0, chunks, step, 0)
        plsc.subcore_barrier()
        pltpu.sync_copy(acc_sh.at[pl.ds(base, rows_per_tile)],
                        out_h.at[cid, pl.ds(base, rows_per_tile)])
        if with_count:
            pltpu.sync_copy(cnt_sh.at[pl.ds(base, rows_per_tile)],
                            cnt_h.at[cid, pl.ds(base, rows_per_tile)])

    return pl.kernel(body, out_type=tuple(out_type), mesh=mesh,
                     scratch_types=scratch,
                     compiler_params=pltpu.CompilerParams(
                         use_tc_tiling_on_sc=False))


def _matmul_tc(x, w, block_rows):
    n, k = x.shape
    m = w.shape[1]
    grid = (n // block_rows,)

    def mm(x_ref, w_ref, o_ref):
        o_ref[...] = jnp.dot(x_ref[...], w_ref[...],
                             preferred_element_type=jnp.float32)

    return pl.pallas_call(
        mm,
        grid=grid,
        in_specs=[pl.BlockSpec((block_rows, k), lambda i: (i, 0)),
                  pl.BlockSpec((k, m), lambda i: (0, 0))],
        out_specs=pl.BlockSpec((block_rows, m), lambda i: (i, 0)),
        out_shape=jax.ShapeDtypeStruct((n, m), jnp.float32),
    )(x, w)


def _mid_tc(parts1, cnt, pre, b1, w2cat, h_dim, c_dim, block_rows):
    n = pre.shape[0]
    grid = (n // block_rows,)

    def mid(p_ref, c_ref, pre_ref, b_ref, w_ref, t2_ref, r2_ref):
        s = p_ref[0] + p_ref[1]
        c = c_ref[0, :, 0:1] + c_ref[1, :, 0:1]
        mean = s / jnp.maximum(c, 1.0)
        h = jax.nn.sigmoid(mean + pre_ref[:, h_dim:2 * h_dim] + b_ref[0:1, :])
        t2r2 = jnp.dot(h, w_ref[...], preferred_element_type=jnp.float32)
        t2_ref[...] = t2r2[:, :c_dim]
        r2_ref[...] = t2r2[:, c_dim:]

    return pl.pallas_call(
        mid,
        grid=grid,
        in_specs=[
            pl.BlockSpec((NC, block_rows, h_dim), lambda i: (0, i, 0)),
            pl.BlockSpec((NC, block_rows, CNTW), lambda i: (0, i, 0)),
            pl.BlockSpec((block_rows, 2 * h_dim), lambda i: (i, 0)),
            pl.BlockSpec((8, h_dim), lambda i: (0, 0)),
            pl.BlockSpec((h_dim, 2 * c_dim), lambda i: (0, 0)),
        ],
        out_specs=[pl.BlockSpec((block_rows, c_dim), lambda i: (i, 0)),
                   pl.BlockSpec((block_rows, c_dim), lambda i: (i, 0))],
        out_shape=[jax.ShapeDtypeStruct((n, c_dim), jnp.float32),
                   jax.ShapeDtypeStruct((n, c_dim), jnp.float32)],
    )(parts1, cnt, pre, b1, w2cat)


def _post_tc(parts2, cnt, r2, b2, c_dim, block_rows):
    n = r2.shape[0]
    grid = (n // block_rows,)

    def post(p_ref, c_ref, r2_ref, b_ref, o_ref):
        s = p_ref[0] + p_ref[1]
        c = c_ref[0, :, 0:1] + c_ref[1, :, 0:1]
        o_ref[...] = s / jnp.maximum(c, 1.0) + r2_ref[...] + b_ref[0:1, :]

    return pl.pallas_call(
        post,
        grid=grid,
        in_specs=[
            pl.BlockSpec((NC, block_rows, c_dim), lambda i: (0, i, 0)),
            pl.BlockSpec((NC, block_rows, CNTW), lambda i: (0, i, 0)),
            pl.BlockSpec((block_rows, c_dim), lambda i: (i, 0)),
            pl.BlockSpec((8, c_dim), lambda i: (0, 0)),
        ],
        out_specs=pl.BlockSpec((block_rows, c_dim), lambda i: (i, 0)),
        out_shape=jax.ShapeDtypeStruct((n, c_dim), jnp.float32),
    )(parts2, cnt, r2, b2)


def kernel(x, edge_index, W1_l, W1_r, b1, W2_l, W2_r, b2):
    n, f_in = x.shape
    h_dim = W1_l.shape[1]
    c_dim = W2_l.shape[1]
    e = edge_index.shape[1]

    per_w = -(-e // NW)
    chunks = -(-per_w // CHUNK)
    e_pad = NW * chunks * CHUNK
    src = edge_index[0].astype(jnp.int32)
    dst = edge_index[1].astype(jnp.int32)
    src = jnp.concatenate(
        [src, jnp.zeros((e_pad - e,), jnp.int32)]).reshape(NW, chunks, CHUNK)
    dst = jnp.concatenate(
        [dst, jnp.full((e_pad - e,), n, jnp.int32)]).reshape(NW, chunks, CHUNK)

    n_pad = ((n + 1 + 8 * NS - 1) // (8 * NS)) * (8 * NS)
    rows_per_tile = n_pad // NS
    xp = jnp.concatenate([x, jnp.zeros((n_pad - n, f_in), x.dtype)], axis=0)

    zeros_f1 = jnp.zeros((rows_per_tile, h_dim), jnp.float32)
    zeros_f2 = jnp.zeros((rows_per_tile, c_dim), jnp.float32)
    zeros_c = jnp.zeros((rows_per_tile, CNTW), jnp.float32)
    ones = jnp.ones((CHUNK, CNTW), jnp.float32)

    w1cat = jnp.concatenate([W1_l, W1_r], axis=1)
    w2cat = jnp.concatenate([W2_l, W2_r], axis=1)
    b1b = jnp.broadcast_to(b1.reshape(1, h_dim), (8, h_dim))
    b2b = jnp.broadcast_to(b2.reshape(1, c_dim), (8, c_dim))

    block_rows = n_pad // 8

    pre = _matmul_tc(xp, w1cat, block_rows=block_rows)
    t1 = pre[:, :h_dim]

    seg1 = _seg_sum_sc(n_pad, rows_per_tile, h_dim, chunks, with_count=True)
    parts1, cnt = seg1(src, dst, t1, zeros_f1, zeros_c, ones)

    t2, r2 = _mid_tc(parts1, cnt, pre, b1b, w2cat, h_dim, c_dim,
                     block_rows=block_rows)

    seg2 = _seg_sum_sc(n_pad, rows_per_tile, c_dim, chunks, with_count=False)
    (parts2,) = seg2(src, dst, t2, zeros_f2)

    out = _post_tc(parts2, cnt, r2, b2b, c_dim, block_rows=block_rows)
    return out[:n]
